# baseline (speedup 1.0000x reference)
import functools

import jax
import jax.numpy as jnp
from jax import lax
from jax.experimental import pallas as pl
from jax.experimental.pallas import tpu as pltpu
from jax.experimental.pallas import tpu_sc as plsc

BATCH = 16384
NFIELD = 20
NPREFIX = 6
VOCAB = 40000
WPAD = 2432
WTOT = 26 * VOCAB + WPAD
SBASE = 239616
SSKEW = NPREFIX * VOCAB - SBASE
WREST = 16 * 50176
WSLICE = WREST // 16
NC, NS, L = 2, 16, 16
NW = NC * NS
RPW = BATCH // NW
RCHUNKS = RPW // L


def _body(prefix_hbm, rest_hbm, w_hbm, bias_hbm, out_hbm,
          stage_ref, idx_ref, vals_ref, acc_ref, pidx_ref, pval_ref,
          wsp_ref, gsem, sem2, sem3, sem4, sem5):
    wid = lax.axis_index("s") * NC + lax.axis_index("c")
    sid = lax.axis_index("s")
    base = wid * RPW
    lanes = lax.iota(jnp.int32, L)

    pidx_ref[...] = jnp.zeros((L,), jnp.int32)
    pltpu.sync_copy(prefix_hbm, pidx_ref.at[pl.ds(0, NPREFIX)])
    pidx_ref[...] = pidx_ref[...] + jnp.where(lanes < NPREFIX,
                                              lanes * VOCAB, 0)

    with jax.named_scope("stage_idx"):
        def stage_row(f, _):
            pltpu.async_copy(rest_hbm.at[f, pl.ds(base, RPW)],
                             idx_ref.at[pl.ds(f * RPW, RPW)], sem4)
            return _
        lax.fori_loop(0, NFIELD, stage_row, 0)

    tstage = pltpu.async_copy(
        w_hbm.at[pl.ds(SBASE + sid * WSLICE, WSLICE)],
        wsp_ref.at[pl.ds(sid * WSLICE, WSLICE)], sem2)
    pgather = pltpu.async_copy(w_hbm.at[pidx_ref], pval_ref, sem3)

    acc_ref[pl.ds(0, L)] = jnp.zeros((L,), jnp.float32)
    bias_d = pltpu.async_copy(bias_hbm, acc_ref.at[pl.ds(0, 1)], sem5)
    with jax.named_scope("scalar"):
        pgather.wait()
        bias_d.wait()
        sb_vec = (jnp.where(lanes < NPREFIX, pval_ref[...], 0.0)
                  + acc_ref[pl.ds(0, L)])
        for k in (1, 2, 4, 8):
            sb_vec = sb_vec + sb_vec.at[lanes ^ k].get(
                mode="promise_in_bounds")

        def initc(c, _):
            acc_ref[pl.ds(c * L, L)] = sb_vec
            return _
        lax.fori_loop(0, RCHUNKS, initc, 0)

    with jax.named_scope("tbl_wait"):
        pltpu.make_async_copy(rest_hbm.at[0, pl.ds(0, NFIELD * RPW)],
                              idx_ref, sem4).wait()
        tstage.wait()
        plsc.subcore_barrier()

    gathers = []
    with jax.named_scope("fire"):
        for f in range(NFIELD):
            wv = wsp_ref.at[pl.ds(SSKEW + f * VOCAB, VOCAB)]
            gathers.append(pltpu.async_copy(
                wv.at[idx_ref.at[pl.ds(f * RPW, RPW)]],
                vals_ref.at[pl.ds(f * RPW, RPW)], gsem.at[f]))

    with jax.named_scope("reduce"):
        for f in range(NFIELD):
            gathers[f].wait()

            def accf(c, _, f=f):
                acc_ref[pl.ds(c * L, L)] = (
                    acc_ref[pl.ds(c * L, L)]
                    + vals_ref[pl.ds(f * RPW + c * L, L)])
                return _
            lax.fori_loop(0, RCHUNKS, accf, 0)

    pltpu.sync_copy(acc_ref, out_hbm.at[pl.ds(base, RPW)])


@jax.jit
def _run(prefix_index, rest_t, w_flat, bias):
    mesh = plsc.VectorSubcoreMesh(core_axis_name="c", subcore_axis_name="s",
                                  num_cores=NC, num_subcores=NS)
    f = pl.kernel(
        _body,
        out_type=jax.ShapeDtypeStruct((BATCH,), jnp.float32),
        mesh=mesh,
        scratch_types=[
            pltpu.VMEM((NFIELD, RPW), jnp.int32),
            pltpu.VMEM((NFIELD * RPW,), jnp.int32),
            pltpu.VMEM((NFIELD * RPW,), jnp.float32),
            pltpu.VMEM((RPW,), jnp.float32),
            pltpu.VMEM((L,), jnp.int32),
            pltpu.VMEM((L,), jnp.float32),
            pltpu.VMEM_SHARED((WREST,), jnp.float32),
            pltpu.SemaphoreType.DMA((NFIELD,)),
            pltpu.SemaphoreType.DMA,
            pltpu.SemaphoreType.DMA,
            pltpu.SemaphoreType.DMA,
            pltpu.SemaphoreType.DMA,
        ],
        compiler_params=pltpu.CompilerParams(needs_layout_passes=False),
    )
    return f(prefix_index, rest_t, w_flat, bias)


def kernel(prefix_index, rest_index, fc_weight, bias):
    rest_t = rest_index.T
    w_flat = jnp.concatenate(
        [fc_weight, jnp.zeros((WPAD, 1), jnp.float32)]).reshape(-1)
    out = _run(prefix_index, rest_t, w_flat, bias)
    return out.reshape(BATCH, 1)


# speedup vs baseline: 1.0185x; 1.0185x over previous
import jax
import jax.numpy as jnp
from jax import lax
from jax.experimental import pallas as pl
from jax.experimental.pallas import tpu as pltpu
from jax.experimental.pallas import tpu_sc as plsc

BATCH = 16384
NFIELD = 20
NPREFIX = 6
VOCAB = 40000
WPAD = 2432
WTOT = 26 * VOCAB + WPAD
SBASE = 239616
SSKEW = NPREFIX * VOCAB - SBASE
WREST = 16 * 50176
WSLICE = WREST // 16
NC, NS, L = 2, 16, 16
NW = NC * NS
RPW = BATCH // NW
RCHUNKS = RPW // L


def _body(prefix_hbm, rest_hbm, w_hbm, bias_hbm, out_hbm,
          idx_ref, vals_ref, acc_ref, pidx_ref, pval_ref,
          wsp_ref, sem, sem2, sem3, sem4, sem5):
    wid = lax.axis_index("s") * NC + lax.axis_index("c")
    sid = lax.axis_index("s")
    base = wid * RPW
    lanes = lax.iota(jnp.int32, L)

    with jax.named_scope("stage_idx"):
        def stage_row(f, _):
            pltpu.async_copy(rest_hbm.at[f, pl.ds(base, RPW)],
                             idx_ref.at[pl.ds(f * RPW, RPW)], sem4)
            return _
        lax.fori_loop(0, NFIELD, stage_row, 0)

    tstage = pltpu.async_copy(
        w_hbm.at[pl.ds(SBASE + sid * WSLICE, WSLICE)],
        wsp_ref.at[pl.ds(sid * WSLICE, WSLICE)], sem2)

    pidx_ref[...] = jnp.zeros((L,), jnp.int32)
    pltpu.sync_copy(prefix_hbm, pidx_ref.at[pl.ds(0, NPREFIX)])
    pidx_ref[...] = pidx_ref[...] + jnp.where(lanes < NPREFIX,
                                              lanes * VOCAB, 0)
    pgather = pltpu.async_copy(w_hbm.at[pidx_ref], pval_ref, sem3)

    acc_ref[pl.ds(0, L)] = jnp.zeros((L,), jnp.float32)
    bias_d = pltpu.async_copy(bias_hbm, acc_ref.at[pl.ds(0, 1)], sem5)
    with jax.named_scope("scalar"):
        pgather.wait()
        bias_d.wait()
        sb_vec = (jnp.where(lanes < NPREFIX, pval_ref[...], 0.0)
                  + acc_ref[pl.ds(0, L)])
        for k in (1, 2, 4, 8):
            sb_vec = sb_vec + sb_vec.at[lanes ^ k].get(
                mode="promise_in_bounds")

    with jax.named_scope("tbl_wait"):
        pltpu.make_async_copy(rest_hbm.at[0, pl.ds(0, NFIELD * RPW)],
                              idx_ref, sem4).wait()
        tstage.wait()
        plsc.subcore_barrier()

    with jax.named_scope("fire"):
        for f in range(NFIELD):
            wv = wsp_ref.at[pl.ds(SSKEW + f * VOCAB, VOCAB)]
            pltpu.async_copy(wv.at[idx_ref.at[pl.ds(f * RPW, RPW)]],
                             vals_ref.at[pl.ds(f * RPW, RPW)], sem)

    with jax.named_scope("drain"):
        pltpu.make_async_copy(w_hbm.at[pl.ds(0, NFIELD * RPW)],
                              vals_ref, sem).wait()

    def row_chunk(c, _):
        acc = sb_vec
        for f in range(NFIELD):
            acc = acc + vals_ref[pl.ds(f * RPW + c * L, L)]
        acc_ref[pl.ds(c * L, L)] = acc
        return _
    with jax.named_scope("reduce"):
        lax.fori_loop(0, RCHUNKS, row_chunk, 0)

    pltpu.sync_copy(acc_ref, out_hbm.at[pl.ds(base, RPW)])


@jax.jit
def _run(prefix_index, rest_t, w_flat, bias):
    mesh = plsc.VectorSubcoreMesh(core_axis_name="c", subcore_axis_name="s",
                                  num_cores=NC, num_subcores=NS)
    f = pl.kernel(
        _body,
        out_type=jax.ShapeDtypeStruct((BATCH,), jnp.float32),
        mesh=mesh,
        scratch_types=[
            pltpu.VMEM((NFIELD * RPW,), jnp.int32),
            pltpu.VMEM((NFIELD * RPW,), jnp.float32),
            pltpu.VMEM((RPW,), jnp.float32),
            pltpu.VMEM((L,), jnp.int32),
            pltpu.VMEM((L,), jnp.float32),
            pltpu.VMEM_SHARED((WREST,), jnp.float32),
            pltpu.SemaphoreType.DMA,
            pltpu.SemaphoreType.DMA,
            pltpu.SemaphoreType.DMA,
            pltpu.SemaphoreType.DMA,
            pltpu.SemaphoreType.DMA,
        ],
        compiler_params=pltpu.CompilerParams(needs_layout_passes=False),
    )
    return f(prefix_index, rest_t, w_flat, bias)


def kernel(prefix_index, rest_index, fc_weight, bias):
    rest_t = rest_index.T
    w_flat = jnp.concatenate(
        [fc_weight, jnp.zeros((WPAD, 1), jnp.float32)]).reshape(-1)
    out = _run(prefix_index, rest_t, w_flat, bias)
    return out.reshape(BATCH, 1)


# speedup vs baseline: 1.0372x; 1.0184x over previous
import jax
import jax.numpy as jnp
from jax import lax
from jax.experimental import pallas as pl
from jax.experimental.pallas import tpu as pltpu
from jax.experimental.pallas import tpu_sc as plsc

BATCH = 16384
NFIELD = 20
NPREFIX = 6
VOCAB = 40000
WPAD = 2432
WTOT = 26 * VOCAB + WPAD
SBASE = 239616
SSKEW = NPREFIX * VOCAB - SBASE
WREST = 16 * 50176
WSLICE = WREST // 16
NC, NS, L = 2, 16, 16
NW = NC * NS
RPW = BATCH // NW
RCHUNKS = RPW // L


def _body(prefix_hbm, rest_hbm, w_hbm, bias_hbm, out_hbm,
          idx_ref, vals_ref, acc_ref, pidx_ref, pval_ref,
          wsp_ref, sem, sem2, sem3, sem4, sem5):
    wid = lax.axis_index("s") * NC + lax.axis_index("c")
    sid = lax.axis_index("s")
    base = wid * RPW
    lanes = lax.iota(jnp.int32, L)

    with jax.named_scope("stage_idx"):
        def stage_row(f, _):
            pltpu.async_copy(rest_hbm.at[f, pl.ds(base, RPW)],
                             idx_ref.at[pl.ds(f * RPW, RPW)], sem4)
            return _
        lax.fori_loop(0, NFIELD, stage_row, 0)

    tstage = pltpu.async_copy(
        w_hbm.at[pl.ds(SBASE + sid * WSLICE, WSLICE)],
        wsp_ref.at[pl.ds(sid * WSLICE, WSLICE)], sem2)

    pidx_ref[...] = jnp.zeros((L,), jnp.int32)
    pltpu.sync_copy(prefix_hbm, pidx_ref.at[pl.ds(0, NPREFIX)])
    pidx_ref[...] = pidx_ref[...] + jnp.where(lanes < NPREFIX,
                                              lanes * VOCAB, 0)
    pgather = pltpu.async_copy(w_hbm.at[pidx_ref], pval_ref, sem3)

    acc_ref[pl.ds(0, L)] = jnp.zeros((L,), jnp.float32)
    bias_d = pltpu.async_copy(bias_hbm, acc_ref.at[pl.ds(0, 1)], sem5)
    with jax.named_scope("scalar"):
        pgather.wait()
        bias_d.wait()
        sb_vec = (jnp.where(lanes < NPREFIX, pval_ref[...], 0.0)
                  + acc_ref[pl.ds(0, L)])
        for k in (1, 2, 4, 8):
            sb_vec = sb_vec + sb_vec.at[lanes ^ k].get(
                mode="promise_in_bounds")

    with jax.named_scope("tbl_wait"):
        pltpu.make_async_copy(rest_hbm.at[0, pl.ds(0, NFIELD * RPW)],
                              idx_ref, sem4).wait()
        tstage.wait()
        plsc.subcore_barrier()

    with jax.named_scope("fire"):
        for f in range(NFIELD):
            wv = wsp_ref.at[pl.ds(SSKEW + f * VOCAB, VOCAB)]
            pltpu.async_copy(wv.at[idx_ref.at[pl.ds(f * RPW, RPW)]],
                             vals_ref.at[pl.ds(f * RPW, RPW)], sem)

    with jax.named_scope("drain"):
        pltpu.make_async_copy(w_hbm.at[pl.ds(0, NFIELD * RPW)],
                              vals_ref, sem).wait()

    def row_chunk(c, _):
        acc0 = sb_vec
        acc1 = sb_vec
        for f in range(NFIELD):
            acc0 = acc0 + vals_ref[pl.ds(f * RPW + 2 * c * L, L)]
            acc1 = acc1 + vals_ref[pl.ds(f * RPW + (2 * c + 1) * L, L)]
        acc_ref[pl.ds(2 * c * L, L)] = acc0
        acc_ref[pl.ds((2 * c + 1) * L, L)] = acc1
        return _
    with jax.named_scope("reduce"):
        lax.fori_loop(0, RCHUNKS // 2, row_chunk, 0)

    pltpu.sync_copy(acc_ref, out_hbm.at[pl.ds(base, RPW)])


@jax.jit
def _run(prefix_index, rest_t, w_flat, bias):
    mesh = plsc.VectorSubcoreMesh(core_axis_name="c", subcore_axis_name="s",
                                  num_cores=NC, num_subcores=NS)
    f = pl.kernel(
        _body,
        out_type=jax.ShapeDtypeStruct((BATCH,), jnp.float32),
        mesh=mesh,
        scratch_types=[
            pltpu.VMEM((NFIELD * RPW,), jnp.int32),
            pltpu.VMEM((NFIELD * RPW,), jnp.float32),
            pltpu.VMEM((RPW,), jnp.float32),
            pltpu.VMEM((L,), jnp.int32),
            pltpu.VMEM((L,), jnp.float32),
            pltpu.VMEM_SHARED((WREST,), jnp.float32),
            pltpu.SemaphoreType.DMA,
            pltpu.SemaphoreType.DMA,
            pltpu.SemaphoreType.DMA,
            pltpu.SemaphoreType.DMA,
            pltpu.SemaphoreType.DMA,
        ],
        compiler_params=pltpu.CompilerParams(needs_layout_passes=False),
    )
    return f(prefix_index, rest_t, w_flat, bias)


def kernel(prefix_index, rest_index, fc_weight, bias):
    rest_t = rest_index.T
    w_flat = jnp.concatenate(
        [fc_weight, jnp.zeros((WPAD, 1), jnp.float32)]).reshape(-1)
    out = _run(prefix_index, rest_t, w_flat, bias)
    return out.reshape(BATCH, 1)
